# baseline (speedup 1.0000x reference)
import jax
import jax.numpy as jnp
import numpy as np
from jax.experimental import pallas as pl
from jax.experimental.pallas import tpu as pltpu

_B, _A, _L = 128, 32, 8192
_R1 = (13, 15, 26, 6)
_R2 = (17, 29, 16, 24)


def _rotl(x, r):
    return jax.lax.shift_left(x, jnp.uint32(r)) | jax.lax.shift_right_logical(
        x, jnp.uint32(32 - r))


def _rounds(x0, x1, rots):
    for r in rots:
        x0 = x0 + x1
        x1 = _rotl(x1, r)
        x1 = x0 ^ x1
    return x0, x1


def _gumbel_tile(idx, k1, k2):
    ks2 = k1 ^ k2 ^ jnp.uint32(0x1BD11BDA)
    x0 = jnp.zeros_like(idx) + k1
    x1 = idx + k2
    x0, x1 = _rounds(x0, x1, _R1)
    x0 = x0 + k2
    x1 = x1 + ks2 + jnp.uint32(1)
    x0, x1 = _rounds(x0, x1, _R2)
    x0 = x0 + ks2
    x1 = x1 + k1 + jnp.uint32(2)
    x0, x1 = _rounds(x0, x1, _R1)
    x0 = x0 + k1
    x1 = x1 + k2 + jnp.uint32(3)
    x0, x1 = _rounds(x0, x1, _R2)
    x0 = x0 + k2
    x1 = x1 + ks2 + jnp.uint32(4)
    x0, x1 = _rounds(x0, x1, _R1)
    x0 = x0 + ks2
    x1 = x1 + k1 + jnp.uint32(5)
    bits = x0 ^ x1
    float_bits = jax.lax.shift_right_logical(bits, jnp.uint32(9)) | jnp.uint32(
        0x3F800000)
    f = jax.lax.bitcast_convert_type(float_bits, jnp.float32) - 1.0
    tiny = jnp.float32(np.finfo(np.float32).tiny)
    u = jax.lax.max(tiny, f * (jnp.float32(1.0) - tiny) + tiny)
    return -jnp.log(-jnp.log(u))


def _mh_body(kd_ref, pos_ref, u_ref, theta_ref, w_ref,
             out_ref, e_ref, acc_ref):
    b = pl.program_id(0)
    t = theta_ref[0]
    w = w_ref[...]

    lane = jax.lax.broadcasted_iota(jnp.int32, t.shape, 1)
    arow = jax.lax.broadcasted_iota(jnp.int32, t.shape, 0)

    idx = (jnp.uint32(b * _A * _L)
           + lane.astype(jnp.uint32) * jnp.uint32(_A)
           + arow.astype(jnp.uint32))
    gt = _gumbel_tile(idx, kd_ref[0], kd_ref[1])

    pos_b = pos_ref[b]
    s = t * 115.0
    score = jnp.where(lane == pos_b, -s, s) + gt

    m = jnp.max(score, axis=0, keepdims=True)
    idxa = jnp.min(jnp.where(score == m, arow, _A), axis=0, keepdims=True)
    newp = jnp.where(arow == idxa, 1.0, 0.0).astype(t.dtype)

    old_e = jnp.sum(t * w)
    new_e = jnp.sum(newp * w)
    acc = u_ref[b] <= (old_e - new_e)

    out_ref[0] = jnp.where(acc, newp, t)
    e_ref[b] = jnp.where(acc, new_e, old_e)
    acc_ref[b] = jnp.where(acc, 1, 0)


def kernel(theta, W):
    B, A, L = theta.shape
    kr = jax.random.key(42)
    k_pos, k_gumbel, k_u = jax.random.split(kr, 3)
    kd = jax.random.key_data(k_gumbel).astype(jnp.uint32)

    pos = jnp.argmin(jax.random.uniform(k_pos, (B, L)), axis=-1)
    pos = pos.astype(jnp.int32)
    u = jnp.log(jax.random.uniform(k_u, (B,), dtype=theta.dtype))

    sample, energy, accept = pl.pallas_call(
        _mh_body,
        grid=(B,),
        in_specs=[
            pl.BlockSpec(memory_space=pltpu.SMEM),
            pl.BlockSpec(memory_space=pltpu.SMEM),
            pl.BlockSpec(memory_space=pltpu.SMEM),
            pl.BlockSpec((1, A, L), lambda b: (b, 0, 0)),
            pl.BlockSpec((A, L), lambda b: (0, 0)),
        ],
        out_specs=[
            pl.BlockSpec((1, A, L), lambda b: (b, 0, 0)),
            pl.BlockSpec(memory_space=pltpu.SMEM),
            pl.BlockSpec(memory_space=pltpu.SMEM),
        ],
        out_shape=[
            jax.ShapeDtypeStruct((B, A, L), theta.dtype),
            jax.ShapeDtypeStruct((B,), theta.dtype),
            jax.ShapeDtypeStruct((B,), jnp.int32),
        ],
    )(kd, pos, u, theta, W)

    return sample, energy, accept.astype(bool)


# speedup vs baseline: 1.1668x; 1.1668x over previous
import jax
import jax.numpy as jnp
import numpy as np
from jax.experimental import pallas as pl
from jax.experimental.pallas import tpu as pltpu

_B, _A, _L = 128, 32, 8192
_R1 = (13, 15, 26, 6)
_R2 = (17, 29, 16, 24)


def _rotl(x, r):
    return jax.lax.shift_left(x, jnp.uint32(r)) | jax.lax.shift_right_logical(
        x, jnp.uint32(32 - r))


def _rounds(x0, x1, rots):
    for r in rots:
        x0 = x0 + x1
        x1 = _rotl(x1, r)
        x1 = x0 ^ x1
    return x0, x1


def _gumbel_tile(idx, k1, k2):
    ks2 = k1 ^ k2 ^ jnp.uint32(0x1BD11BDA)
    x0 = jnp.zeros_like(idx) + k1
    x1 = idx + k2
    x0, x1 = _rounds(x0, x1, _R1)
    x0 = x0 + k2
    x1 = x1 + ks2 + jnp.uint32(1)
    x0, x1 = _rounds(x0, x1, _R2)
    x0 = x0 + ks2
    x1 = x1 + k1 + jnp.uint32(2)
    x0, x1 = _rounds(x0, x1, _R1)
    x0 = x0 + k1
    x1 = x1 + k2 + jnp.uint32(3)
    x0, x1 = _rounds(x0, x1, _R2)
    x0 = x0 + k2
    x1 = x1 + ks2 + jnp.uint32(4)
    x0, x1 = _rounds(x0, x1, _R1)
    x0 = x0 + ks2
    x1 = x1 + k1 + jnp.uint32(5)
    bits = x0 ^ x1
    float_bits = jax.lax.shift_right_logical(bits, jnp.uint32(9)) | jnp.uint32(
        0x3F800000)
    f = jax.lax.bitcast_convert_type(float_bits, jnp.float32) - 1.0
    tiny = jnp.float32(np.finfo(np.float32).tiny)
    u = jax.lax.max(tiny, f * (jnp.float32(1.0) - tiny) + tiny)
    return -jnp.log(-jnp.log(u))


_CH = 512
_NCH = _L // _CH


def _mh_body(kd_ref, pos_ref, u_ref, theta_ref, w_ref,
             out_ref, e_ref, acc_ref, newp_ref):
    b = pl.program_id(0)
    pos_b = pos_ref[b]
    k1 = kd_ref[0]
    k2 = kd_ref[1]

    def chunk(i, carry):
        old_p, new_p = carry
        t = theta_ref[0, :, pl.ds(i * _CH, _CH)]
        w = w_ref[:, pl.ds(i * _CH, _CH)]
        lane = jax.lax.broadcasted_iota(jnp.int32, t.shape, 1) + i * _CH
        arow = jax.lax.broadcasted_iota(jnp.int32, t.shape, 0)

        idx = (jnp.uint32(b * _A * _L)
               + lane.astype(jnp.uint32) * jnp.uint32(_A)
               + arow.astype(jnp.uint32))
        gt = _gumbel_tile(idx, k1, k2)

        s = t * 115.0
        score = jnp.where(lane == pos_b, -s, s) + gt

        m = jnp.max(score, axis=0, keepdims=True)
        idxa = jnp.min(jnp.where(score == m, arow, _A), axis=0, keepdims=True)
        newp = jnp.where(arow == idxa, 1.0, 0.0).astype(t.dtype)
        newp_ref[:, pl.ds(i * _CH, _CH)] = newp

        return (old_p + jnp.sum(t * w), new_p + jnp.sum(newp * w))

    old_e, new_e = jax.lax.fori_loop(
        0, _NCH, chunk, (jnp.float32(0.0), jnp.float32(0.0)))

    acc = u_ref[b] <= (old_e - new_e)
    out_ref[0] = jnp.where(acc, newp_ref[...], theta_ref[0])
    e_ref[b] = jnp.where(acc, new_e, old_e)
    acc_ref[b] = jnp.where(acc, 1, 0)


def kernel(theta, W):
    B, A, L = theta.shape
    kr = jax.random.key(42)
    k_pos, k_gumbel, k_u = jax.random.split(kr, 3)
    kd = jax.random.key_data(k_gumbel).astype(jnp.uint32)

    pos = jnp.argmin(jax.random.uniform(k_pos, (B, L)), axis=-1)
    pos = pos.astype(jnp.int32)
    u = jnp.log(jax.random.uniform(k_u, (B,), dtype=theta.dtype))

    sample, energy, accept = pl.pallas_call(
        _mh_body,
        grid=(B,),
        in_specs=[
            pl.BlockSpec(memory_space=pltpu.SMEM),
            pl.BlockSpec(memory_space=pltpu.SMEM),
            pl.BlockSpec(memory_space=pltpu.SMEM),
            pl.BlockSpec((1, A, L), lambda b: (b, 0, 0)),
            pl.BlockSpec((A, L), lambda b: (0, 0)),
        ],
        out_specs=[
            pl.BlockSpec((1, A, L), lambda b: (b, 0, 0)),
            pl.BlockSpec(memory_space=pltpu.SMEM),
            pl.BlockSpec(memory_space=pltpu.SMEM),
        ],
        out_shape=[
            jax.ShapeDtypeStruct((B, A, L), theta.dtype),
            jax.ShapeDtypeStruct((B,), theta.dtype),
            jax.ShapeDtypeStruct((B,), jnp.int32),
        ],
        scratch_shapes=[pltpu.VMEM((A, L), theta.dtype)],
    )(kd, pos, u, theta, W)

    return sample, energy, accept.astype(bool)


# speedup vs baseline: 1.3480x; 1.1553x over previous
import jax
import jax.numpy as jnp
from jax.experimental import pallas as pl
from jax.experimental.pallas import tpu as pltpu

_B, _A, _L = 128, 32, 8192


def _bf16_quant(x):
    u = jax.lax.bitcast_convert_type(x, jnp.uint32)
    r = u + jnp.uint32(0x7FFF) + (jax.lax.shift_right_logical(
        u, jnp.uint32(16)) & jnp.uint32(1))
    return jax.lax.bitcast_convert_type(r & jnp.uint32(0xFFFF0000), jnp.float32)


def _mh_body(pos_ref, u_ref, theta_ref, g_ref, w_ref,
             out_ref, e_ref, acc_ref):
    b = pl.program_id(0)
    t = theta_ref[0]
    w = w_ref[...]
    gt = g_ref[0]

    pos_b = pos_ref[b]
    lane = jax.lax.broadcasted_iota(jnp.int32, t.shape, 1)
    arow = jax.lax.broadcasted_iota(jnp.int32, t.shape, 0)

    s = t * 115.0
    score = jnp.where(lane == pos_b, -s, s) + gt

    m = jnp.max(score, axis=0, keepdims=True)
    idx = jnp.min(jnp.where(score == m, arow, _A), axis=0, keepdims=True)
    newp = jnp.where(arow == idx, 1.0, 0.0).astype(t.dtype)

    tq = _bf16_quant(t)
    old_e = jnp.sum(tq * w)
    new_e = jnp.sum(newp * w)
    acc = u_ref[b] <= (old_e - new_e)

    out_ref[0] = jnp.where(acc, newp, t)
    e_ref[b] = jnp.where(acc, new_e, old_e)
    acc_ref[b] = jnp.where(acc, 1, 0)


def kernel(theta, W):
    B, A, L = theta.shape
    kr = jax.random.key(42)
    k_pos, k_gumbel, k_u = jax.random.split(kr, 3)

    pos = jnp.argmin(jax.random.uniform(k_pos, (B, L)), axis=-1)
    pos = pos.astype(jnp.int32)
    g = jnp.swapaxes(jax.random.gumbel(k_gumbel, (B, L, A), dtype=theta.dtype),
                     1, 2)
    u = jnp.log(jax.random.uniform(k_u, (B,), dtype=theta.dtype))
    Wq = _bf16_quant(W)

    sample, energy, accept = pl.pallas_call(
        _mh_body,
        grid=(B,),
        in_specs=[
            pl.BlockSpec(memory_space=pltpu.SMEM),
            pl.BlockSpec(memory_space=pltpu.SMEM),
            pl.BlockSpec((1, A, L), lambda b: (b, 0, 0)),
            pl.BlockSpec((1, A, L), lambda b: (b, 0, 0)),
            pl.BlockSpec((A, L), lambda b: (0, 0)),
        ],
        out_specs=[
            pl.BlockSpec((1, A, L), lambda b: (b, 0, 0)),
            pl.BlockSpec(memory_space=pltpu.SMEM),
            pl.BlockSpec(memory_space=pltpu.SMEM),
        ],
        out_shape=[
            jax.ShapeDtypeStruct((B, A, L), theta.dtype),
            jax.ShapeDtypeStruct((B,), theta.dtype),
            jax.ShapeDtypeStruct((B,), jnp.int32),
        ],
    )(pos, u, theta, g, Wq)

    return sample, energy, accept.astype(bool)


# speedup vs baseline: 1.4254x; 1.0574x over previous
import jax
import jax.numpy as jnp
from jax.experimental import pallas as pl
from jax.experimental.pallas import tpu as pltpu

_B, _A, _L = 128, 32, 8192


def _diag_sum(p):
    r = jax.lax.broadcasted_iota(jnp.int32, p.shape, 0)
    c = jax.lax.broadcasted_iota(jnp.int32, p.shape, 1)
    return jnp.sum(jnp.where(r == c, p, 0.0))


def _mh_body(pos_ref, u_ref, theta_ref, g_ref, w_ref,
             out_ref, e_ref, acc_ref):
    b = pl.program_id(0)
    t = theta_ref[0]
    w = w_ref[...]
    gt = g_ref[0]

    pos_b = pos_ref[b]
    lane = jax.lax.broadcasted_iota(jnp.int32, t.shape, 1)
    arow = jax.lax.broadcasted_iota(jnp.int32, t.shape, 0)

    s = t * 115.0
    score = jnp.where(lane == pos_b, -s, s) + gt

    m = jnp.max(score, axis=0, keepdims=True)
    idx = jnp.min(jnp.where(score == m, arow, _A), axis=0, keepdims=True)
    newp = jnp.where(arow == idx, 1.0, 0.0).astype(t.dtype)

    dn = (((1,), (1,)), ((), ()))
    tb = t.astype(jnp.bfloat16)
    npb = newp.astype(jnp.bfloat16)
    old_e = _diag_sum(jax.lax.dot_general(
        tb, w, dn, preferred_element_type=jnp.float32))
    new_e = _diag_sum(jax.lax.dot_general(
        npb, w, dn, preferred_element_type=jnp.float32))
    acc = u_ref[b] <= (old_e - new_e)

    out_ref[0] = jnp.where(acc, newp, t)
    e_ref[b] = jnp.where(acc, new_e, old_e)
    acc_ref[b] = jnp.where(acc, 1, 0)


def kernel(theta, W):
    B, A, L = theta.shape
    kr = jax.random.key(42)
    k_pos, k_gumbel, k_u = jax.random.split(kr, 3)

    pos = jnp.argmin(jax.random.uniform(k_pos, (B, L)), axis=-1)
    pos = pos.astype(jnp.int32)
    g = jnp.swapaxes(jax.random.gumbel(k_gumbel, (B, L, A), dtype=theta.dtype),
                     1, 2)
    u = jnp.log(jax.random.uniform(k_u, (B,), dtype=theta.dtype))
    Wq = W.astype(jnp.bfloat16)

    sample, energy, accept = pl.pallas_call(
        _mh_body,
        grid=(B,),
        in_specs=[
            pl.BlockSpec(memory_space=pltpu.SMEM),
            pl.BlockSpec(memory_space=pltpu.SMEM),
            pl.BlockSpec((1, A, L), lambda b: (b, 0, 0)),
            pl.BlockSpec((1, A, L), lambda b: (b, 0, 0)),
            pl.BlockSpec((A, L), lambda b: (0, 0)),
        ],
        out_specs=[
            pl.BlockSpec((1, A, L), lambda b: (b, 0, 0)),
            pl.BlockSpec(memory_space=pltpu.SMEM),
            pl.BlockSpec(memory_space=pltpu.SMEM),
        ],
        out_shape=[
            jax.ShapeDtypeStruct((B, A, L), theta.dtype),
            jax.ShapeDtypeStruct((B,), theta.dtype),
            jax.ShapeDtypeStruct((B,), jnp.int32),
        ],
    )(pos, u, theta, g, Wq)

    return sample, energy, accept.astype(bool)


# speedup vs baseline: 1.5062x; 1.0567x over previous
import jax
import jax.numpy as jnp
from jax.experimental import pallas as pl
from jax.experimental.pallas import tpu as pltpu

_B, _A, _L = 128, 32, 8192


def _diag_sum(p):
    r = jax.lax.broadcasted_iota(jnp.int32, p.shape, 0)
    c = jax.lax.broadcasted_iota(jnp.int32, p.shape, 1)
    return jnp.sum(jnp.where(r == c, p, 0.0))


_CPB = 2


def _mh_body(pos_ref, u_ref, theta_ref, g_ref, w_ref,
             out_ref, e_ref, acc_ref):
    for c in range(_CPB):
        _mh_chain(c, pos_ref, u_ref, theta_ref, g_ref, w_ref,
                  out_ref, e_ref, acc_ref)


def _mh_chain(c, pos_ref, u_ref, theta_ref, g_ref, w_ref,
              out_ref, e_ref, acc_ref):
    b = pl.program_id(0) * _CPB + c
    t = theta_ref[c]
    w = w_ref[...]
    gt = g_ref[c]

    pos_b = pos_ref[b]
    lane = jax.lax.broadcasted_iota(jnp.int32, t.shape, 1)
    arow = jax.lax.broadcasted_iota(jnp.int32, t.shape, 0)

    s = t * 115.0
    score = jnp.where(lane == pos_b, -s, s) + gt

    m = jnp.max(score, axis=0, keepdims=True)
    idx = jnp.min(jnp.where(score == m, arow, _A), axis=0, keepdims=True)
    newp = jnp.where(arow == idx, 1.0, 0.0).astype(t.dtype)

    dn = (((1,), (1,)), ((), ()))
    tb = t.astype(jnp.bfloat16)
    npb = newp.astype(jnp.bfloat16)
    old_e = _diag_sum(jax.lax.dot_general(
        tb, w, dn, preferred_element_type=jnp.float32))
    new_e = _diag_sum(jax.lax.dot_general(
        npb, w, dn, preferred_element_type=jnp.float32))
    acc = u_ref[b] <= (old_e - new_e)

    out_ref[c] = jnp.where(acc, newp, t)
    e_ref[b] = jnp.where(acc, new_e, old_e)
    acc_ref[b] = jnp.where(acc, 1, 0)


def kernel(theta, W):
    B, A, L = theta.shape
    kr = jax.random.key(42)
    k_pos, k_gumbel, k_u = jax.random.split(kr, 3)

    pos = jnp.argmin(jax.random.uniform(k_pos, (B, L)), axis=-1)
    pos = pos.astype(jnp.int32)
    g = jnp.swapaxes(jax.random.gumbel(k_gumbel, (B, L, A), dtype=theta.dtype),
                     1, 2)
    u = jnp.log(jax.random.uniform(k_u, (B,), dtype=theta.dtype))
    Wq = W.astype(jnp.bfloat16)

    sample, energy, accept = pl.pallas_call(
        _mh_body,
        grid=(B // _CPB,),
        in_specs=[
            pl.BlockSpec(memory_space=pltpu.SMEM),
            pl.BlockSpec(memory_space=pltpu.SMEM),
            pl.BlockSpec((_CPB, A, L), lambda b: (b, 0, 0)),
            pl.BlockSpec((_CPB, A, L), lambda b: (b, 0, 0)),
            pl.BlockSpec((A, L), lambda b: (0, 0)),
        ],
        out_specs=[
            pl.BlockSpec((_CPB, A, L), lambda b: (b, 0, 0)),
            pl.BlockSpec(memory_space=pltpu.SMEM),
            pl.BlockSpec(memory_space=pltpu.SMEM),
        ],
        out_shape=[
            jax.ShapeDtypeStruct((B, A, L), theta.dtype),
            jax.ShapeDtypeStruct((B,), theta.dtype),
            jax.ShapeDtypeStruct((B,), jnp.int32),
        ],
    )(pos, u, theta, g, Wq)

    return sample, energy, accept.astype(bool)


# speedup vs baseline: 1.5439x; 1.0251x over previous
import jax
import jax.numpy as jnp
from jax.experimental import pallas as pl
from jax.experimental.pallas import tpu as pltpu

_B, _A, _L = 128, 32, 8192


def _diag_sum(p):
    r = jax.lax.broadcasted_iota(jnp.int32, p.shape, 0)
    c = jax.lax.broadcasted_iota(jnp.int32, p.shape, 1)
    return jnp.sum(jnp.where(r == c, p, 0.0))


_CPB = 4


def _mh_body(pos_ref, u_ref, theta_ref, g_ref, w_ref,
             out_ref, e_ref, acc_ref):
    for c in range(_CPB):
        _mh_chain(c, pos_ref, u_ref, theta_ref, g_ref, w_ref,
                  out_ref, e_ref, acc_ref)


def _mh_chain(c, pos_ref, u_ref, theta_ref, g_ref, w_ref,
              out_ref, e_ref, acc_ref):
    b = pl.program_id(0) * _CPB + c
    t = theta_ref[c]
    w = w_ref[...]
    gt = g_ref[c]

    pos_b = pos_ref[b]
    lane = jax.lax.broadcasted_iota(jnp.int32, t.shape, 1)
    arow = jax.lax.broadcasted_iota(jnp.int32, t.shape, 0)

    s = t * 115.0
    score = jnp.where(lane == pos_b, -s, s) + gt

    m = jnp.max(score, axis=0, keepdims=True)
    idx = jnp.min(jnp.where(score == m, arow, _A), axis=0, keepdims=True)
    newp = jnp.where(arow == idx, 1.0, 0.0).astype(t.dtype)

    dn = (((1,), (1,)), ((), ()))
    tb = t.astype(jnp.bfloat16)
    npb = newp.astype(jnp.bfloat16)
    old_e = _diag_sum(jax.lax.dot_general(
        tb, w, dn, preferred_element_type=jnp.float32))
    new_e = _diag_sum(jax.lax.dot_general(
        npb, w, dn, preferred_element_type=jnp.float32))
    acc = u_ref[b] <= (old_e - new_e)

    out_ref[c] = jnp.where(acc, newp, t)
    e_ref[b] = jnp.where(acc, new_e, old_e)
    acc_ref[b] = jnp.where(acc, 1, 0)


def kernel(theta, W):
    B, A, L = theta.shape
    kr = jax.random.key(42)
    k_pos, k_gumbel, k_u = jax.random.split(kr, 3)

    pos = jnp.argmin(jax.random.uniform(k_pos, (B, L)), axis=-1)
    pos = pos.astype(jnp.int32)
    g = jnp.swapaxes(jax.random.gumbel(k_gumbel, (B, L, A), dtype=theta.dtype),
                     1, 2)
    u = jnp.log(jax.random.uniform(k_u, (B,), dtype=theta.dtype))
    Wq = W.astype(jnp.bfloat16)

    sample, energy, accept = pl.pallas_call(
        _mh_body,
        grid=(B // _CPB,),
        in_specs=[
            pl.BlockSpec(memory_space=pltpu.SMEM),
            pl.BlockSpec(memory_space=pltpu.SMEM),
            pl.BlockSpec((_CPB, A, L), lambda b: (b, 0, 0)),
            pl.BlockSpec((_CPB, A, L), lambda b: (b, 0, 0)),
            pl.BlockSpec((A, L), lambda b: (0, 0)),
        ],
        out_specs=[
            pl.BlockSpec((_CPB, A, L), lambda b: (b, 0, 0)),
            pl.BlockSpec(memory_space=pltpu.SMEM),
            pl.BlockSpec(memory_space=pltpu.SMEM),
        ],
        out_shape=[
            jax.ShapeDtypeStruct((B, A, L), theta.dtype),
            jax.ShapeDtypeStruct((B,), theta.dtype),
            jax.ShapeDtypeStruct((B,), jnp.int32),
        ],
    )(pos, u, theta, g, Wq)

    return sample, energy, accept.astype(bool)
